# baseline (speedup 1.0000x reference)
import functools

import jax
import jax.numpy as jnp
from jax import lax
from jax.experimental import pallas as pl
from jax.experimental.pallas import tpu as pltpu
from jax.experimental.pallas import tpu_sc as plsc

F32 = jnp.float32
_NC = 2
_NS = 16
_L = 16
_C = 40


def _bcast_lane(v, h):
    idx = jnp.full((_L, 1), h, jnp.int32)
    dn = lax.GatherDimensionNumbers(
        offset_dims=(), collapsed_slice_dims=(0,), start_index_map=(0,))
    return lax.gather(v, idx, dn, slice_sizes=(1,),
                      mode=lax.GatherScatterMode.PROMISE_IN_BOUNDS)


def _edge_pass(tsrc, tdst, eix, zeros_hbm):
    n = tsrc.shape[0]
    nchunks_tot = eix.shape[1]
    nw = _NC * _NS
    cpw = nchunks_tot // nw
    rpt = (n // _NS) // 8 * 8
    rem = n - rpt * _NS
    mesh = plsc.VectorSubcoreMesh(core_axis_name="c", subcore_axis_name="s",
                                  num_cores=_NC, num_subcores=_NS)

    @functools.partial(
        pl.kernel,
        out_type=jax.ShapeDtypeStruct((_NC, n, 144), F32),
        mesh=mesh,
        compiler_params=pltpu.CompilerParams(use_tc_tiling_on_sc=False),
        scratch_types=[
            pltpu.VMEM((2, cpw, _C), jnp.int32),
            pltpu.VMEM((_C, 144), F32),
            pltpu.VMEM((_C, 144), F32),
            pltpu.VMEM((_C, 16), F32),
            pltpu.VMEM((_C, 16), F32),
            pltpu.VMEM((_C, 144), F32),
            pltpu.VMEM_SHARED((n, 144), F32),
            pltpu.SemaphoreType.DMA,
            pltpu.SemaphoreType.DMA,
            pltpu.SemaphoreType.DMA,
            pltpu.SemaphoreType.DMA,
            pltpu.SemaphoreType.DMA,
        ],
    )
    def k(tsrc_hbm, tdst_hbm, eix_hbm, z_hbm, out_hbm,
          idxb, g0, g1, d0, d1, ob, acc,
          sgs0, sgs1, sgd0, sgd1, ssc):
        c = lax.axis_index("c")
        s = lax.axis_index("s")
        wid = c * _NS + s
        gslot = (g0, g1)
        dslot = (d0, d1)
        gsem = (sgs0, sgs1)
        dsem = (sgd0, sgd1)

        cpi = pltpu.make_async_copy(
            eix_hbm.at[:, pl.ds(wid * cpw, cpw), :], idxb, sgs0)
        cpi.start()

        pltpu.sync_copy(z_hbm.at[pl.ds(s * rpt, rpt)],
                        acc.at[pl.ds(s * rpt, rpt)])

        @pl.when(s == 0)
        def _():
            pltpu.sync_copy(z_hbm.at[pl.ds(rpt * _NS, rem)],
                            acc.at[pl.ds(rpt * _NS, rem)])

        cpi.wait()
        plsc.subcore_barrier()

        def gather_desc(chunk, slot):
            cg = pltpu.make_async_copy(
                tsrc_hbm.at[idxb.at[0, chunk]], gslot[slot], gsem[slot])
            cd = pltpu.make_async_copy(
                tdst_hbm.at[idxb.at[1, chunk]], dslot[slot], dsem[slot])
            return cg, cd

        def issue_gather(chunk, slot):
            cg, cd = gather_desc(chunk, slot)
            cg.start()
            cd.start()

        def wait_gather(chunk, slot):
            cg, cd = gather_desc(chunk, slot)
            cg.wait()
            cd.wait()

        def compute(slot):
            g, d = gslot[slot], dslot[slot]

            @plsc.parallel_loop(0, _C, 1, unroll=8)
            def _(ei):
                a = g[ei, pl.ds(128, _L)] + d[ei, pl.ds(0, _L)]
                a = jnp.maximum(a, a * 0.2)
                ex = jnp.exp(a)
                ob[ei, pl.ds(128, _L)] = ex
                for h in range(8):
                    bh = _bcast_lane(ex, h)
                    ob[ei, pl.ds(h * _L, _L)] = g[ei, pl.ds(h * _L, _L)] * bh

        def scatter_desc(chunk):
            return pltpu.make_async_copy(
                ob, acc.at[idxb.at[1, chunk]], ssc)

        issue_gather(0, 0)
        issue_gather(1, 1)

        wait_gather(0, 0)
        compute(0)
        scatter_desc(0).start(add=True)
        issue_gather(2, 0)

        @pl.loop(1, cpw)
        def _(chunk):
            slot = lax.rem(chunk, 2)

            @pl.when(slot == 0)
            def _():
                wait_gather(chunk, 0)

            @pl.when(slot == 1)
            def _():
                wait_gather(chunk, 1)

            scatter_desc(chunk - 1).wait()

            @pl.when(slot == 0)
            def _():
                compute(0)

            @pl.when(slot == 1)
            def _():
                compute(1)

            scatter_desc(chunk).start(add=True)

            @pl.when((slot == 0) & (chunk + 2 < cpw))
            def _():
                issue_gather(chunk + 2, 0)

            @pl.when((slot == 1) & (chunk + 2 < cpw))
            def _():
                issue_gather(chunk + 2, 1)

        scatter_desc(cpw - 1).wait()

        plsc.subcore_barrier()
        pltpu.sync_copy(acc.at[pl.ds(s * rpt, rpt)],
                        out_hbm.at[c, pl.ds(s * rpt, rpt)])

        @pl.when(s == 0)
        def _():
            pltpu.sync_copy(acc.at[pl.ds(rpt * _NS, rem)],
                            out_hbm.at[c, pl.ds(rpt * _NS, rem)])

    return k(tsrc, tdst, eix, zeros_hbm)


def _prep_tc(x, w_ext, w_dst):
    n = x.shape[0]

    def body(x_ref, we_ref, wd_ref, ts_ref, td_ref):
        xx = x_ref[...]
        ts_ref[...] = jnp.dot(xx, we_ref[...], preferred_element_type=F32)
        td_ref[...] = jnp.dot(xx, wd_ref[...], preferred_element_type=F32)

    return pl.pallas_call(
        body,
        out_shape=(jax.ShapeDtypeStruct((n, 144), F32),
                   jax.ShapeDtypeStruct((n, 16), F32)),
    )(x, w_ext, w_dst)


def _mid_tc(acc, b1, w_ext, w_dst, p8):
    n = acc.shape[1]

    def body(acc_ref, b1_ref, we_ref, wd_ref, p8_ref, ts_ref, td_ref):
        a = acc_ref[0] + acc_ref[1]
        num = a[:, :128]
        den = a[:, 128:136]
        r = 1.0 / (den + 1e-16)
        rex = jnp.dot(r, p8_ref[...], preferred_element_type=F32)
        hpre = num * rex + b1_ref[...]
        hh = jnp.where(hpre > 0, hpre, jnp.exp(hpre) - 1.0)
        ts_ref[...] = jnp.dot(hh, we_ref[...], preferred_element_type=F32)
        td_ref[...] = jnp.dot(hh, wd_ref[...], preferred_element_type=F32)

    return pl.pallas_call(
        body,
        out_shape=(jax.ShapeDtypeStruct((n, 144), F32),
                   jax.ShapeDtypeStruct((n, 16), F32)),
    )(acc, b1, w_ext, w_dst, p8)


def _final_tc(acc, b2, p0):
    n = acc.shape[1]

    def body(acc_ref, b2_ref, p0_ref, o_ref):
        a = acc_ref[0] + acc_ref[1]
        num = a[:, :128]
        den = a[:, 128:136]
        r = 1.0 / (den + 1e-16)
        rex = jnp.dot(r, p0_ref[...], preferred_element_type=F32)
        o = num * rex + b2_ref[...]
        m = jnp.max(o, axis=1, keepdims=True)
        z = o - m
        lse = jnp.log(jnp.sum(jnp.exp(z), axis=1, keepdims=True))
        o_ref[...] = z - lse

    return pl.pallas_call(
        body,
        out_shape=jax.ShapeDtypeStruct((n, 128), F32),
    )(acc, b2, p0)


def kernel(x, edge_index, W1, att_src1, att_dst1, b1, W2, att_src2,
           att_dst2, b2):
    n = x.shape[0]
    heads, hid = att_src1.shape
    eix = edge_index.reshape(2, -1, _C)

    eye = jnp.eye(heads, dtype=F32)
    a_s = (eye[:, None, :] * att_src1[:, :, None]).reshape(heads * hid, heads)
    a_d = (eye[:, None, :] * att_dst1[:, :, None]).reshape(heads * hid, heads)
    z8 = jnp.zeros((heads * hid, 8), F32)
    w1_ext = jnp.concatenate([W1, W1 @ a_s, z8], axis=1)
    w1_dst = jnp.concatenate([W1 @ a_d, z8], axis=1)
    w2_ext = jnp.concatenate(
        [W2, jnp.tile((W2 @ att_src2[0])[:, None], (1, 16))], axis=1)
    w2_dst = jnp.tile((W2 @ att_dst2[0])[:, None], (1, 16))
    p8 = (jnp.arange(128)[None, :] // hid
          == jnp.arange(heads)[:, None]).astype(F32)
    p0 = jnp.concatenate([jnp.ones((1, 128), F32),
                          jnp.zeros((7, 128), F32)], axis=0)
    zeros_hbm = jnp.zeros((n, 144), F32)

    tsrc1, tdst1 = _prep_tc(x, w1_ext, w1_dst)
    acc1 = _edge_pass(tsrc1, tdst1, eix, zeros_hbm)
    tsrc2, tdst2 = _mid_tc(acc1, b1.reshape(1, -1), w2_ext, w2_dst, p8)
    acc2 = _edge_pass(tsrc2, tdst2, eix, zeros_hbm)
    return _final_tc(acc2, b2.reshape(1, -1), p0)


# speedup vs baseline: 1.0065x; 1.0065x over previous
import functools

import jax
import jax.numpy as jnp
from jax import lax
from jax.experimental import pallas as pl
from jax.experimental.pallas import tpu as pltpu
from jax.experimental.pallas import tpu_sc as plsc

F32 = jnp.float32
_NC = 2
_NS = 16
_L = 16
_C = 40


def _bcast_lane(v, h):
    idx = jnp.full((_L, 1), h, jnp.int32)
    dn = lax.GatherDimensionNumbers(
        offset_dims=(), collapsed_slice_dims=(0,), start_index_map=(0,))
    return lax.gather(v, idx, dn, slice_sizes=(1,),
                      mode=lax.GatherScatterMode.PROMISE_IN_BOUNDS)


def _edge_pass(tsrc, tdst, eix, zeros_hbm):
    n = tsrc.shape[0]
    nchunks_tot = eix.shape[1]
    nw = _NC * _NS
    cpw = nchunks_tot // nw
    rpt = (n // _NS) // 8 * 8
    rem = n - rpt * _NS
    mesh = plsc.VectorSubcoreMesh(core_axis_name="c", subcore_axis_name="s",
                                  num_cores=_NC, num_subcores=_NS)

    @functools.partial(
        pl.kernel,
        out_type=jax.ShapeDtypeStruct((_NC, n, 144), F32),
        mesh=mesh,
        compiler_params=pltpu.CompilerParams(use_tc_tiling_on_sc=False),
        scratch_types=[
            pltpu.VMEM((2, cpw, _C), jnp.int32),
            pltpu.VMEM((_C, 144), F32),
            pltpu.VMEM((_C, 144), F32),
            pltpu.VMEM((_C, 16), F32),
            pltpu.VMEM((_C, 16), F32),
            pltpu.VMEM((_C, 144), F32),
            pltpu.VMEM_SHARED((n, 144), F32),
            pltpu.SemaphoreType.DMA,
            pltpu.SemaphoreType.DMA,
            pltpu.SemaphoreType.DMA,
            pltpu.SemaphoreType.DMA,
            pltpu.SemaphoreType.DMA,
        ],
    )
    def k(tsrc_hbm, tdst_hbm, eix_hbm, z_hbm, out_hbm,
          idxb, g0, g1, d0, d1, ob, acc,
          sgs0, sgs1, sgd0, sgd1, ssc):
        c = lax.axis_index("c")
        s = lax.axis_index("s")
        wid = c * _NS + s
        gslot = (g0, g1)
        dslot = (d0, d1)
        gsem = (sgs0, sgs1)
        dsem = (sgd0, sgd1)

        cpi = pltpu.make_async_copy(
            eix_hbm.at[:, pl.ds(wid * cpw, cpw), :], idxb, sgs0)
        cpi.start()

        pltpu.sync_copy(z_hbm.at[pl.ds(s * rpt, rpt)],
                        acc.at[pl.ds(s * rpt, rpt)])

        @pl.when(s == 0)
        def _():
            pltpu.sync_copy(z_hbm.at[pl.ds(rpt * _NS, rem)],
                            acc.at[pl.ds(rpt * _NS, rem)])

        cpi.wait()
        plsc.subcore_barrier()

        def gather_desc(chunk, slot):
            cg = pltpu.make_async_copy(
                tsrc_hbm.at[idxb.at[0, chunk]], gslot[slot], gsem[slot])
            cd = pltpu.make_async_copy(
                tdst_hbm.at[idxb.at[1, chunk]], dslot[slot], dsem[slot])
            return cg, cd

        def issue_gather(chunk, slot):
            cg, cd = gather_desc(chunk, slot)
            cg.start()
            cd.start()

        def wait_gather(chunk, slot):
            cg, cd = gather_desc(chunk, slot)
            cg.wait()
            cd.wait()

        def compute(slot):
            g, d = gslot[slot], dslot[slot]

            @plsc.parallel_loop(0, _C, 1, unroll=4)
            def _(ei):
                a = g[ei, pl.ds(128, _L)] + d[ei, pl.ds(0, _L)]
                a = jnp.maximum(a, a * 0.2)
                ex = jnp.exp(a)
                ob[ei, pl.ds(128, _L)] = ex
                for h in range(8):
                    bh = _bcast_lane(ex, h)
                    ob[ei, pl.ds(h * _L, _L)] = g[ei, pl.ds(h * _L, _L)] * bh

        def scatter_desc(chunk):
            return pltpu.make_async_copy(
                ob, acc.at[idxb.at[1, chunk]], ssc)

        issue_gather(0, 0)
        issue_gather(1, 1)

        wait_gather(0, 0)
        compute(0)
        scatter_desc(0).start(add=True)
        issue_gather(2, 0)

        @pl.loop(1, cpw)
        def _(chunk):
            slot = lax.rem(chunk, 2)

            @pl.when(slot == 0)
            def _():
                wait_gather(chunk, 0)

            @pl.when(slot == 1)
            def _():
                wait_gather(chunk, 1)

            scatter_desc(chunk - 1).wait()

            @pl.when(slot == 0)
            def _():
                compute(0)

            @pl.when(slot == 1)
            def _():
                compute(1)

            scatter_desc(chunk).start(add=True)

            @pl.when((slot == 0) & (chunk + 2 < cpw))
            def _():
                issue_gather(chunk + 2, 0)

            @pl.when((slot == 1) & (chunk + 2 < cpw))
            def _():
                issue_gather(chunk + 2, 1)

        scatter_desc(cpw - 1).wait()

        plsc.subcore_barrier()
        pltpu.sync_copy(acc.at[pl.ds(s * rpt, rpt)],
                        out_hbm.at[c, pl.ds(s * rpt, rpt)])

        @pl.when(s == 0)
        def _():
            pltpu.sync_copy(acc.at[pl.ds(rpt * _NS, rem)],
                            out_hbm.at[c, pl.ds(rpt * _NS, rem)])

    return k(tsrc, tdst, eix, zeros_hbm)


def _prep_tc(x, w_ext, w_dst):
    n = x.shape[0]

    def body(x_ref, we_ref, wd_ref, ts_ref, td_ref):
        xx = x_ref[...]
        ts_ref[...] = jnp.dot(xx, we_ref[...], preferred_element_type=F32)
        td_ref[...] = jnp.dot(xx, wd_ref[...], preferred_element_type=F32)

    return pl.pallas_call(
        body,
        out_shape=(jax.ShapeDtypeStruct((n, 144), F32),
                   jax.ShapeDtypeStruct((n, 16), F32)),
    )(x, w_ext, w_dst)


def _mid_tc(acc, b1, w_ext, w_dst, p8):
    n = acc.shape[1]

    def body(acc_ref, b1_ref, we_ref, wd_ref, p8_ref, ts_ref, td_ref):
        a = acc_ref[0] + acc_ref[1]
        num = a[:, :128]
        den = a[:, 128:136]
        r = 1.0 / (den + 1e-16)
        rex = jnp.dot(r, p8_ref[...], preferred_element_type=F32)
        hpre = num * rex + b1_ref[...]
        hh = jnp.where(hpre > 0, hpre, jnp.exp(hpre) - 1.0)
        ts_ref[...] = jnp.dot(hh, we_ref[...], preferred_element_type=F32)
        td_ref[...] = jnp.dot(hh, wd_ref[...], preferred_element_type=F32)

    return pl.pallas_call(
        body,
        out_shape=(jax.ShapeDtypeStruct((n, 144), F32),
                   jax.ShapeDtypeStruct((n, 16), F32)),
    )(acc, b1, w_ext, w_dst, p8)


def _final_tc(acc, b2, p0):
    n = acc.shape[1]

    def body(acc_ref, b2_ref, p0_ref, o_ref):
        a = acc_ref[0] + acc_ref[1]
        num = a[:, :128]
        den = a[:, 128:136]
        r = 1.0 / (den + 1e-16)
        rex = jnp.dot(r, p0_ref[...], preferred_element_type=F32)
        o = num * rex + b2_ref[...]
        m = jnp.max(o, axis=1, keepdims=True)
        z = o - m
        lse = jnp.log(jnp.sum(jnp.exp(z), axis=1, keepdims=True))
        o_ref[...] = z - lse

    return pl.pallas_call(
        body,
        out_shape=jax.ShapeDtypeStruct((n, 128), F32),
    )(acc, b2, p0)


def kernel(x, edge_index, W1, att_src1, att_dst1, b1, W2, att_src2,
           att_dst2, b2):
    n = x.shape[0]
    heads, hid = att_src1.shape
    eix = edge_index.reshape(2, -1, _C)

    eye = jnp.eye(heads, dtype=F32)
    a_s = (eye[:, None, :] * att_src1[:, :, None]).reshape(heads * hid, heads)
    a_d = (eye[:, None, :] * att_dst1[:, :, None]).reshape(heads * hid, heads)
    z8 = jnp.zeros((heads * hid, 8), F32)
    w1_ext = jnp.concatenate([W1, W1 @ a_s, z8], axis=1)
    w1_dst = jnp.concatenate([W1 @ a_d, z8], axis=1)
    w2_ext = jnp.concatenate(
        [W2, jnp.tile((W2 @ att_src2[0])[:, None], (1, 16))], axis=1)
    w2_dst = jnp.tile((W2 @ att_dst2[0])[:, None], (1, 16))
    p8 = (jnp.arange(128)[None, :] // hid
          == jnp.arange(heads)[:, None]).astype(F32)
    p0 = jnp.concatenate([jnp.ones((1, 128), F32),
                          jnp.zeros((7, 128), F32)], axis=0)
    zeros_hbm = jnp.zeros((n, 144), F32)

    tsrc1, tdst1 = _prep_tc(x, w1_ext, w1_dst)
    acc1 = _edge_pass(tsrc1, tdst1, eix, zeros_hbm)
    tsrc2, tdst2 = _mid_tc(acc1, b1.reshape(1, -1), w2_ext, w2_dst, p8)
    acc2 = _edge_pass(tsrc2, tdst2, eix, zeros_hbm)
    return _final_tc(acc2, b2.reshape(1, -1), p0)


# speedup vs baseline: 1.0789x; 1.0720x over previous
import functools

import jax
import jax.numpy as jnp
from jax import lax
from jax.experimental import pallas as pl
from jax.experimental.pallas import tpu as pltpu
from jax.experimental.pallas import tpu_sc as plsc

F32 = jnp.float32
_NC = 2
_NS = 16
_L = 16
_C = 40


def _bcast_lane(v, h):
    idx = jnp.full((_L, 1), h, jnp.int32)
    dn = lax.GatherDimensionNumbers(
        offset_dims=(), collapsed_slice_dims=(0,), start_index_map=(0,))
    return lax.gather(v, idx, dn, slice_sizes=(1,),
                      mode=lax.GatherScatterMode.PROMISE_IN_BOUNDS)


def _edge_pass(xw, tasrc, tadst, eix, zeros_hbm):
    n = xw.shape[0]
    nchunks_tot = eix.shape[1]
    nw = _NC * _NS
    cpw = nchunks_tot // nw
    rpt = (n // _NS) // 8 * 8
    rem = n - rpt * _NS
    mesh = plsc.VectorSubcoreMesh(core_axis_name="c", subcore_axis_name="s",
                                  num_cores=_NC, num_subcores=_NS)

    @functools.partial(
        pl.kernel,
        out_type=(jax.ShapeDtypeStruct((_NC, n, 128), F32),
                  jax.ShapeDtypeStruct((_NC, n, 16), F32)),
        mesh=mesh,
        compiler_params=pltpu.CompilerParams(use_tc_tiling_on_sc=False),
        scratch_types=[
            pltpu.VMEM((2, cpw, _C), jnp.int32),
            pltpu.VMEM((_C, 128), F32),
            pltpu.VMEM((_C, 128), F32),
            pltpu.VMEM((_C, 16), F32),
            pltpu.VMEM((_C, 16), F32),
            pltpu.VMEM((_C, 16), F32),
            pltpu.VMEM((_C, 16), F32),
            pltpu.VMEM((_C, 144), F32),
            pltpu.VMEM_SHARED((n, 144), F32),
            pltpu.SemaphoreType.DMA,
            pltpu.SemaphoreType.DMA,
            pltpu.SemaphoreType.DMA,
            pltpu.SemaphoreType.DMA,
            pltpu.SemaphoreType.DMA,
            pltpu.SemaphoreType.DMA,
            pltpu.SemaphoreType.DMA,
        ],
    )
    def k(xw_hbm, tasrc_hbm, tadst_hbm, eix_hbm, z_hbm, outn_hbm, outd_hbm,
          idxb, g0, g1, as0, as1, ad0, ad1, ob, acc,
          sgs0, sgs1, sas0, sas1, sad0, sad1, ssc):
        c = lax.axis_index("c")
        s = lax.axis_index("s")
        wid = c * _NS + s
        gslot = (g0, g1)
        aslot = (as0, as1)
        dslot = (ad0, ad1)
        gsem = (sgs0, sgs1)
        asem = (sas0, sas1)
        dsem = (sad0, sad1)

        cpi = pltpu.make_async_copy(
            eix_hbm.at[:, pl.ds(wid * cpw, cpw), :], idxb, sgs0)
        cpi.start()

        pltpu.sync_copy(z_hbm.at[pl.ds(s * rpt, rpt)],
                        acc.at[pl.ds(s * rpt, rpt)])

        @pl.when(s == 0)
        def _():
            pltpu.sync_copy(z_hbm.at[pl.ds(rpt * _NS, rem)],
                            acc.at[pl.ds(rpt * _NS, rem)])

        cpi.wait()
        plsc.subcore_barrier()

        def gather_desc(chunk, slot):
            cg = pltpu.make_async_copy(
                xw_hbm.at[idxb.at[0, chunk]], gslot[slot], gsem[slot])
            ca = pltpu.make_async_copy(
                tasrc_hbm.at[idxb.at[0, chunk]], aslot[slot], asem[slot])
            cd = pltpu.make_async_copy(
                tadst_hbm.at[idxb.at[1, chunk]], dslot[slot], dsem[slot])
            return cg, ca, cd

        def issue_gather(chunk, slot):
            for cp in gather_desc(chunk, slot):
                cp.start()

        def wait_gather(chunk, slot):
            for cp in gather_desc(chunk, slot):
                cp.wait()

        def compute(slot):
            g, av, d = gslot[slot], aslot[slot], dslot[slot]

            @plsc.parallel_loop(0, _C, 1, unroll=4)
            def _(ei):
                a = av[ei, pl.ds(0, _L)] + d[ei, pl.ds(0, _L)]
                a = jnp.maximum(a, a * 0.2)
                ex = jnp.exp(a)
                ob[ei, pl.ds(128, _L)] = ex
                for h in range(8):
                    bh = _bcast_lane(ex, h)
                    ob[ei, pl.ds(h * _L, _L)] = g[ei, pl.ds(h * _L, _L)] * bh

        def scatter_desc(chunk):
            return pltpu.make_async_copy(
                ob, acc.at[idxb.at[1, chunk]], ssc)

        issue_gather(0, 0)
        issue_gather(1, 1)

        wait_gather(0, 0)
        compute(0)
        scatter_desc(0).start(add=True)
        issue_gather(2, 0)

        @pl.loop(1, cpw)
        def _(chunk):
            slot = lax.rem(chunk, 2)

            @pl.when(slot == 0)
            def _():
                wait_gather(chunk, 0)

            @pl.when(slot == 1)
            def _():
                wait_gather(chunk, 1)

            scatter_desc(chunk - 1).wait()

            @pl.when(slot == 0)
            def _():
                compute(0)

            @pl.when(slot == 1)
            def _():
                compute(1)

            scatter_desc(chunk).start(add=True)

            @pl.when((slot == 0) & (chunk + 2 < cpw))
            def _():
                issue_gather(chunk + 2, 0)

            @pl.when((slot == 1) & (chunk + 2 < cpw))
            def _():
                issue_gather(chunk + 2, 1)

        scatter_desc(cpw - 1).wait()

        plsc.subcore_barrier()
        pltpu.sync_copy(acc.at[pl.ds(s * rpt, rpt), pl.ds(0, 128)],
                        outn_hbm.at[c, pl.ds(s * rpt, rpt)])
        pltpu.sync_copy(acc.at[pl.ds(s * rpt, rpt), pl.ds(128, 16)],
                        outd_hbm.at[c, pl.ds(s * rpt, rpt)])

        @pl.when(s == 0)
        def _():
            pltpu.sync_copy(acc.at[pl.ds(rpt * _NS, rem), pl.ds(0, 128)],
                            outn_hbm.at[c, pl.ds(rpt * _NS, rem)])
            pltpu.sync_copy(acc.at[pl.ds(rpt * _NS, rem), pl.ds(128, 16)],
                            outd_hbm.at[c, pl.ds(rpt * _NS, rem)])

    return k(xw, tasrc, tadst, eix, zeros_hbm)


def _prep_tc(x, w1, w_src, w_dst):
    n = x.shape[0]

    def body(x_ref, w_ref, ws_ref, wd_ref, xw_ref, ts_ref, td_ref):
        xx = x_ref[...]
        xw_ref[...] = jnp.dot(xx, w_ref[...], preferred_element_type=F32)
        ts_ref[...] = jnp.dot(xx, ws_ref[...], preferred_element_type=F32)
        td_ref[...] = jnp.dot(xx, wd_ref[...], preferred_element_type=F32)

    return pl.pallas_call(
        body,
        out_shape=(jax.ShapeDtypeStruct((n, 128), F32),
                   jax.ShapeDtypeStruct((n, 16), F32),
                   jax.ShapeDtypeStruct((n, 16), F32)),
    )(x, w1, w_src, w_dst)


def _mid_tc(num, den, b1, w2, w_src, w_dst, p8):
    n = num.shape[1]

    def body(num_ref, den_ref, b1_ref, w_ref, ws_ref, wd_ref, p8_ref,
             xw_ref, ts_ref, td_ref):
        d = den_ref[0, :, :8] + den_ref[1, :, :8]
        r = 1.0 / (d + 1e-16)
        rex = jnp.dot(r, p8_ref[...], preferred_element_type=F32)
        hpre = (num_ref[0] + num_ref[1]) * rex + b1_ref[...]
        hh = jnp.where(hpre > 0, hpre, jnp.exp(hpre) - 1.0)
        xw_ref[...] = jnp.dot(hh, w_ref[...], preferred_element_type=F32)
        ts_ref[...] = jnp.dot(hh, ws_ref[...], preferred_element_type=F32)
        td_ref[...] = jnp.dot(hh, wd_ref[...], preferred_element_type=F32)

    return pl.pallas_call(
        body,
        out_shape=(jax.ShapeDtypeStruct((n, 128), F32),
                   jax.ShapeDtypeStruct((n, 16), F32),
                   jax.ShapeDtypeStruct((n, 16), F32)),
    )(num, den, b1, w2, w_src, w_dst, p8)


def _final_tc(num, den, b2):
    n = num.shape[1]

    def body(num_ref, den_ref, b2_ref, o_ref):
        d = den_ref[0, :, :1] + den_ref[1, :, :1]
        o = (num_ref[0] + num_ref[1]) / (d + 1e-16) + b2_ref[...]
        m = jnp.max(o, axis=1, keepdims=True)
        z = o - m
        lse = jnp.log(jnp.sum(jnp.exp(z), axis=1, keepdims=True))
        o_ref[...] = z - lse

    return pl.pallas_call(
        body,
        out_shape=jax.ShapeDtypeStruct((n, 128), F32),
    )(num, den, b2)


def kernel(x, edge_index, W1, att_src1, att_dst1, b1, W2, att_src2,
           att_dst2, b2):
    n = x.shape[0]
    heads, hid = att_src1.shape
    eix = edge_index.reshape(2, -1, _C)

    eye = jnp.eye(heads, dtype=F32)
    a_s = (eye[:, None, :] * att_src1[:, :, None]).reshape(heads * hid, heads)
    a_d = (eye[:, None, :] * att_dst1[:, :, None]).reshape(heads * hid, heads)
    z8 = jnp.zeros((heads * hid, 8), F32)
    w1_src = jnp.concatenate([W1 @ a_s, z8], axis=1)
    w1_dst = jnp.concatenate([W1 @ a_d, z8], axis=1)
    w2_src = jnp.tile((W2 @ att_src2[0])[:, None], (1, 16))
    w2_dst = jnp.tile((W2 @ att_dst2[0])[:, None], (1, 16))
    p8 = (jnp.arange(128)[None, :] // hid
          == jnp.arange(heads)[:, None]).astype(F32)
    zeros_hbm = jnp.zeros((n, 144), F32)

    xw1, ts1, td1 = _prep_tc(x, W1, w1_src, w1_dst)
    num1, den1 = _edge_pass(xw1, ts1, td1, eix, zeros_hbm)
    xw2, ts2, td2 = _mid_tc(num1, den1, b1.reshape(1, -1), W2,
                            w2_src, w2_dst, p8)
    num2, den2 = _edge_pass(xw2, ts2, td2, eix, zeros_hbm)
    return _final_tc(num2, den2, b2.reshape(1, -1))


# speedup vs baseline: 1.1001x; 1.0196x over previous
import functools

import numpy as np

import jax
import jax.numpy as jnp
from jax import lax
from jax.experimental import pallas as pl
from jax.experimental.pallas import tpu as pltpu
from jax.experimental.pallas import tpu_sc as plsc

F32 = jnp.float32
_NC = 2
_NS = 16
_L = 16
_C = 40


def _bcast_lane(v, h):
    idx = jnp.full((_L, 1), h, jnp.int32)
    dn = lax.GatherDimensionNumbers(
        offset_dims=(), collapsed_slice_dims=(0,), start_index_map=(0,))
    return lax.gather(v, idx, dn, slice_sizes=(1,),
                      mode=lax.GatherScatterMode.PROMISE_IN_BOUNDS)


def _rot8(v):
    idx = jnp.reshape(lax.rem(lax.iota(jnp.int32, _L),
                              jnp.int32(8)) + jnp.int32(8), (_L, 1))
    dn = lax.GatherDimensionNumbers(
        offset_dims=(), collapsed_slice_dims=(0,), start_index_map=(0,))
    return lax.gather(v, idx, dn, slice_sizes=(1,),
                      mode=lax.GatherScatterMode.PROMISE_IN_BOUNDS)


def _edge_pass(xw, ta, eix, zeros_hbm):
    n = xw.shape[0]
    nchunks_tot = eix.shape[1]
    nw = _NC * _NS
    cpw = nchunks_tot // nw
    rpt = (n // _NS) // 8 * 8
    rem = n - rpt * _NS
    mesh = plsc.VectorSubcoreMesh(core_axis_name="c", subcore_axis_name="s",
                                  num_cores=_NC, num_subcores=_NS)

    @functools.partial(
        pl.kernel,
        out_type=(jax.ShapeDtypeStruct((_NC, n, 128), F32),
                  jax.ShapeDtypeStruct((_NC, n, 16), F32)),
        mesh=mesh,
        compiler_params=pltpu.CompilerParams(use_tc_tiling_on_sc=False),
        scratch_types=[
            pltpu.VMEM((2, cpw, _C), jnp.int32),
            pltpu.VMEM((_C, 128), F32),
            pltpu.VMEM((_C, 128), F32),
            pltpu.VMEM((_C, 16), F32),
            pltpu.VMEM((_C, 16), F32),
            pltpu.VMEM((_C, 16), F32),
            pltpu.VMEM((_C, 16), F32),
            pltpu.VMEM((_C, 144), F32),
            pltpu.VMEM_SHARED((n, 144), F32),
            pltpu.SemaphoreType.DMA,
            pltpu.SemaphoreType.DMA,
            pltpu.SemaphoreType.DMA,
            pltpu.SemaphoreType.DMA,
            pltpu.SemaphoreType.DMA,
            pltpu.SemaphoreType.DMA,
            pltpu.SemaphoreType.DMA,
        ],
    )
    def k(xw_hbm, ta_hbm, eix_hbm, z_hbm, outn_hbm, outd_hbm,
          idxb, g0, g1, as0, as1, ad0, ad1, ob, acc,
          sgs0, sgs1, sas0, sas1, sad0, sad1, ssc):
        c = lax.axis_index("c")
        s = lax.axis_index("s")
        wid = c * _NS + s
        gslot = (g0, g1)
        aslot = (as0, as1)
        dslot = (ad0, ad1)
        gsem = (sgs0, sgs1)
        asem = (sas0, sas1)
        dsem = (sad0, sad1)

        cpi = pltpu.make_async_copy(
            eix_hbm.at[:, pl.ds(wid * cpw, cpw), :], idxb, sgs0)
        cpi.start()

        pltpu.sync_copy(z_hbm.at[pl.ds(s * rpt, rpt)],
                        acc.at[pl.ds(s * rpt, rpt)])

        @pl.when(s == 0)
        def _():
            pltpu.sync_copy(z_hbm.at[pl.ds(rpt * _NS, rem)],
                            acc.at[pl.ds(rpt * _NS, rem)])

        cpi.wait()
        plsc.subcore_barrier()

        def gather_desc(chunk, slot):
            cg = pltpu.make_async_copy(
                xw_hbm.at[idxb.at[0, chunk]], gslot[slot], gsem[slot])
            ca = pltpu.make_async_copy(
                ta_hbm.at[idxb.at[0, chunk]], aslot[slot], asem[slot])
            cd = pltpu.make_async_copy(
                ta_hbm.at[idxb.at[1, chunk]], dslot[slot], dsem[slot])
            return cg, ca, cd

        def issue_gather(chunk, slot):
            for cp in gather_desc(chunk, slot):
                cp.start()

        def wait_gather(chunk, slot):
            for cp in gather_desc(chunk, slot):
                cp.wait()

        def compute(slot):
            g, av, d = gslot[slot], aslot[slot], dslot[slot]

            @plsc.parallel_loop(0, _C, 1, unroll=4)
            def _(ei):
                a = av[ei, pl.ds(0, _L)] + _rot8(d[ei, pl.ds(0, _L)])
                a = jnp.maximum(a, a * 0.2)
                ex = jnp.exp(a)
                ob[ei, pl.ds(128, _L)] = ex
                for h in range(8):
                    bh = _bcast_lane(ex, h)
                    ob[ei, pl.ds(h * _L, _L)] = g[ei, pl.ds(h * _L, _L)] * bh

        def scatter_desc(chunk):
            return pltpu.make_async_copy(
                ob, acc.at[idxb.at[1, chunk]], ssc)

        issue_gather(0, 0)
        issue_gather(1, 1)

        wait_gather(0, 0)
        compute(0)
        scatter_desc(0).start(add=True)
        issue_gather(2, 0)

        @pl.loop(1, cpw)
        def _(chunk):
            slot = lax.rem(chunk, 2)

            @pl.when(slot == 0)
            def _():
                wait_gather(chunk, 0)

            @pl.when(slot == 1)
            def _():
                wait_gather(chunk, 1)

            scatter_desc(chunk - 1).wait()

            @pl.when(slot == 0)
            def _():
                compute(0)

            @pl.when(slot == 1)
            def _():
                compute(1)

            scatter_desc(chunk).start(add=True)

            @pl.when((slot == 0) & (chunk + 2 < cpw))
            def _():
                issue_gather(chunk + 2, 0)

            @pl.when((slot == 1) & (chunk + 2 < cpw))
            def _():
                issue_gather(chunk + 2, 1)

        scatter_desc(cpw - 1).wait()

        plsc.subcore_barrier()
        pltpu.sync_copy(acc.at[pl.ds(s * rpt, rpt), pl.ds(0, 128)],
                        outn_hbm.at[c, pl.ds(s * rpt, rpt)])
        pltpu.sync_copy(acc.at[pl.ds(s * rpt, rpt), pl.ds(128, 16)],
                        outd_hbm.at[c, pl.ds(s * rpt, rpt)])

        @pl.when(s == 0)
        def _():
            pltpu.sync_copy(acc.at[pl.ds(rpt * _NS, rem), pl.ds(0, 128)],
                            outn_hbm.at[c, pl.ds(rpt * _NS, rem)])
            pltpu.sync_copy(acc.at[pl.ds(rpt * _NS, rem), pl.ds(128, 16)],
                            outd_hbm.at[c, pl.ds(rpt * _NS, rem)])

    return k(xw, ta, eix, zeros_hbm)


def _prep_tc(x, w1, w_a):
    n = x.shape[0]

    def body(x_ref, w_ref, wa_ref, xw_ref, ta_ref):
        xx = x_ref[...]
        xw_ref[...] = jnp.dot(xx, w_ref[...], preferred_element_type=F32)
        ta_ref[...] = jnp.dot(xx, wa_ref[...], preferred_element_type=F32)

    return pl.pallas_call(
        body,
        out_shape=(jax.ShapeDtypeStruct((n, 128), F32),
                   jax.ShapeDtypeStruct((n, 16), F32)),
    )(x, w1, w_a)


def _mid_tc(num, den, b1, w2, w_a, p8):
    n = num.shape[1]

    def body(num_ref, den_ref, b1_ref, w_ref, wa_ref, p8_ref,
             xw_ref, ta_ref):
        d = den_ref[0, :, :8] + den_ref[1, :, :8]
        r = 1.0 / (d + 1e-16)
        rex = jnp.dot(r, p8_ref[...], preferred_element_type=F32)
        hpre = (num_ref[0] + num_ref[1]) * rex + b1_ref[...]
        hh = jnp.where(hpre > 0, hpre, jnp.exp(hpre) - 1.0)
        xw_ref[...] = jnp.dot(hh, w_ref[...], preferred_element_type=F32)
        ta_ref[...] = jnp.dot(hh, wa_ref[...], preferred_element_type=F32)

    return pl.pallas_call(
        body,
        out_shape=(jax.ShapeDtypeStruct((n, 128), F32),
                   jax.ShapeDtypeStruct((n, 16), F32)),
    )(num, den, b1, w2, w_a, p8)


def _final_tc(num, den, b2):
    n = num.shape[1]

    def body(num_ref, den_ref, b2_ref, o_ref):
        d = den_ref[0, :, :1] + den_ref[1, :, :1]
        o = (num_ref[0] + num_ref[1]) / (d + 1e-16) + b2_ref[...]
        m = jnp.max(o, axis=1, keepdims=True)
        z = o - m
        lse = jnp.log(jnp.sum(jnp.exp(z), axis=1, keepdims=True))
        o_ref[...] = z - lse

    return pl.pallas_call(
        body,
        out_shape=jax.ShapeDtypeStruct((n, 128), F32),
    )(num, den, b2)


def kernel(x, edge_index, W1, att_src1, att_dst1, b1, W2, att_src2,
           att_dst2, b2):
    n = x.shape[0]
    heads, hid = att_src1.shape
    eix = edge_index.reshape(2, -1, _C)

    eye = jnp.eye(heads, dtype=F32)
    a_s = (eye[:, None, :] * att_src1[:, :, None]).reshape(heads * hid, heads)
    a_d = (eye[:, None, :] * att_dst1[:, :, None]).reshape(heads * hid, heads)
    w1_a = jnp.concatenate([W1 @ a_s, W1 @ a_d], axis=1)
    w2_a = jnp.concatenate(
        [jnp.tile((W2 @ att_src2[0])[:, None], (1, 8)),
         jnp.tile((W2 @ att_dst2[0])[:, None], (1, 8))], axis=1)
    p8 = (jnp.arange(128)[None, :] // hid
          == jnp.arange(heads)[:, None]).astype(F32)
    zeros_hbm = jnp.zeros((n, 144), F32)

    xw1, ta1 = _prep_tc(x, W1, w1_a)
    num1, den1 = _edge_pass(xw1, ta1, eix, zeros_hbm)
    xw2, ta2 = _mid_tc(num1, den1, b1.reshape(1, -1), W2, w2_a, p8)
    num2, den2 = _edge_pass(xw2, ta2, eix, zeros_hbm)
    return _final_tc(num2, den2, b2.reshape(1, -1))


# speedup vs baseline: 1.2408x; 1.1279x over previous
import functools

import numpy as np

import jax
import jax.numpy as jnp
from jax import lax
from jax.experimental import pallas as pl
from jax.experimental.pallas import tpu as pltpu
from jax.experimental.pallas import tpu_sc as plsc

F32 = jnp.float32
_NC = 2
_NS = 16
_L = 16
_C = 80


def _bcast_lane(v, h):
    idx = jnp.full((_L, 1), h, jnp.int32)
    dn = lax.GatherDimensionNumbers(
        offset_dims=(), collapsed_slice_dims=(0,), start_index_map=(0,))
    return lax.gather(v, idx, dn, slice_sizes=(1,),
                      mode=lax.GatherScatterMode.PROMISE_IN_BOUNDS)


def _rot8(v):
    idx = jnp.reshape(lax.rem(lax.iota(jnp.int32, _L),
                              jnp.int32(8)) + jnp.int32(8), (_L, 1))
    dn = lax.GatherDimensionNumbers(
        offset_dims=(), collapsed_slice_dims=(0,), start_index_map=(0,))
    return lax.gather(v, idx, dn, slice_sizes=(1,),
                      mode=lax.GatherScatterMode.PROMISE_IN_BOUNDS)


def _edge_pass(xw, ta, eix, zeros_hbm):
    n = xw.shape[0]
    nchunks_tot = eix.shape[1]
    nw = _NC * _NS
    cpw = nchunks_tot // nw
    rpt = (n // _NS) // 8 * 8
    rem = n - rpt * _NS
    mesh = plsc.VectorSubcoreMesh(core_axis_name="c", subcore_axis_name="s",
                                  num_cores=_NC, num_subcores=_NS)

    @functools.partial(
        pl.kernel,
        out_type=(jax.ShapeDtypeStruct((_NC, n, 128), F32),
                  jax.ShapeDtypeStruct((_NC, n, 16), F32)),
        mesh=mesh,
        compiler_params=pltpu.CompilerParams(use_tc_tiling_on_sc=False),
        scratch_types=[
            pltpu.VMEM((4, 2, _C), jnp.int32),
            pltpu.VMEM((_C, 128), F32),
            pltpu.VMEM((_C, 128), F32),
            pltpu.VMEM((_C, 16), F32),
            pltpu.VMEM((_C, 16), F32),
            pltpu.VMEM((_C, 16), F32),
            pltpu.VMEM((_C, 16), F32),
            pltpu.VMEM((_C, 144), F32),
            pltpu.VMEM_SHARED((n, 144), F32),
            pltpu.SemaphoreType.DMA,
            pltpu.SemaphoreType.DMA,
            pltpu.SemaphoreType.DMA,
            pltpu.SemaphoreType.DMA,
            pltpu.SemaphoreType.DMA,
            pltpu.SemaphoreType.DMA,
            pltpu.SemaphoreType.DMA,
            pltpu.SemaphoreType.DMA,
        ],
    )
    def k(xw_hbm, ta_hbm, eix_hbm, z_hbm, outn_hbm, outd_hbm,
          idxb, g0, g1, as0, as1, ad0, ad1, ob, acc,
          sgs0, sgs1, sas0, sas1, sad0, sad1, ssc, sidx):
        c = lax.axis_index("c")
        s = lax.axis_index("s")
        wid = c * _NS + s
        gslot = (g0, g1)
        aslot = (as0, as1)
        dslot = (ad0, ad1)
        gsem = (sgs0, sgs1)
        asem = (sas0, sas1)
        dsem = (sad0, sad1)

        pltpu.sync_copy(z_hbm.at[pl.ds(s * rpt, rpt)],
                        acc.at[pl.ds(s * rpt, rpt)])

        @pl.when(s == 0)
        def _():
            pltpu.sync_copy(z_hbm.at[pl.ds(rpt * _NS, rem)],
                            acc.at[pl.ds(rpt * _NS, rem)])

        plsc.subcore_barrier()

        def idx_desc(chunk):
            q = lax.rem(chunk, 4)
            return pltpu.make_async_copy(
                eix_hbm.at[:, wid * cpw + chunk, :], idxb.at[q], sidx)

        def gather_desc(chunk, slot):
            q = lax.rem(chunk, 4)
            cg = pltpu.make_async_copy(
                xw_hbm.at[idxb.at[q, 0]], gslot[slot], gsem[slot])
            ca = pltpu.make_async_copy(
                ta_hbm.at[idxb.at[q, 0]], aslot[slot], asem[slot])
            cd = pltpu.make_async_copy(
                ta_hbm.at[idxb.at[q, 1]], dslot[slot], dsem[slot])
            return cg, ca, cd

        def issue_gather(chunk, slot):
            for cp in gather_desc(chunk, slot):
                cp.start()

        def wait_gather(chunk, slot):
            for cp in gather_desc(chunk, slot):
                cp.wait()

        def compute(slot):
            g, av, d = gslot[slot], aslot[slot], dslot[slot]

            @plsc.parallel_loop(0, _C, 1, unroll=4)
            def _(ei):
                a = av[ei, pl.ds(0, _L)] + _rot8(d[ei, pl.ds(0, _L)])
                a = jnp.maximum(a, a * 0.2)
                ex = jnp.exp(a)
                ob[ei, pl.ds(128, _L)] = ex
                for h in range(8):
                    bh = _bcast_lane(ex, h)
                    ob[ei, pl.ds(h * _L, _L)] = g[ei, pl.ds(h * _L, _L)] * bh

        def scatter_desc(chunk):
            q = lax.rem(chunk, 4)
            return pltpu.make_async_copy(
                ob, acc.at[idxb.at[q, 1]], ssc)

        for q0 in range(4):
            idx_desc(q0).start()
        idx_desc(0).wait()
        issue_gather(0, 0)
        idx_desc(1).wait()
        issue_gather(1, 1)

        wait_gather(0, 0)
        compute(0)
        scatter_desc(0).start(add=True)
        idx_desc(2).wait()
        issue_gather(2, 0)

        @pl.loop(1, cpw)
        def _(chunk):
            slot = lax.rem(chunk, 2)

            @pl.when(slot == 0)
            def _():
                wait_gather(chunk, 0)

            @pl.when(slot == 1)
            def _():
                wait_gather(chunk, 1)

            scatter_desc(chunk - 1).wait()

            @pl.when(chunk + 3 < cpw)
            def _():
                idx_desc(chunk + 3).start()

            @pl.when(slot == 0)
            def _():
                compute(0)

            @pl.when(slot == 1)
            def _():
                compute(1)

            scatter_desc(chunk).start(add=True)

            @pl.when((slot == 0) & (chunk + 2 < cpw))
            def _():
                idx_desc(chunk + 2).wait()
                issue_gather(chunk + 2, 0)

            @pl.when((slot == 1) & (chunk + 2 < cpw))
            def _():
                idx_desc(chunk + 2).wait()
                issue_gather(chunk + 2, 1)

        scatter_desc(cpw - 1).wait()

        plsc.subcore_barrier()
        pltpu.sync_copy(acc.at[pl.ds(s * rpt, rpt), pl.ds(0, 128)],
                        outn_hbm.at[c, pl.ds(s * rpt, rpt)])
        pltpu.sync_copy(acc.at[pl.ds(s * rpt, rpt), pl.ds(128, 16)],
                        outd_hbm.at[c, pl.ds(s * rpt, rpt)])

        @pl.when(s == 0)
        def _():
            pltpu.sync_copy(acc.at[pl.ds(rpt * _NS, rem), pl.ds(0, 128)],
                            outn_hbm.at[c, pl.ds(rpt * _NS, rem)])
            pltpu.sync_copy(acc.at[pl.ds(rpt * _NS, rem), pl.ds(128, 16)],
                            outd_hbm.at[c, pl.ds(rpt * _NS, rem)])

    return k(xw, ta, eix, zeros_hbm)


def _prep_tc(x, w1, w_a):
    n = x.shape[0]

    def body(x_ref, w_ref, wa_ref, xw_ref, ta_ref):
        xx = x_ref[...]
        xw_ref[...] = jnp.dot(xx, w_ref[...], preferred_element_type=F32)
        ta_ref[...] = jnp.dot(xx, wa_ref[...], preferred_element_type=F32)

    return pl.pallas_call(
        body,
        out_shape=(jax.ShapeDtypeStruct((n, 128), F32),
                   jax.ShapeDtypeStruct((n, 16), F32)),
    )(x, w1, w_a)


def _mid_tc(num, den, b1, w2, w_a, p8):
    n = num.shape[1]

    def body(num_ref, den_ref, b1_ref, w_ref, wa_ref, p8_ref,
             xw_ref, ta_ref):
        d = den_ref[0, :, :8] + den_ref[1, :, :8]
        r = 1.0 / (d + 1e-16)
        rex = jnp.dot(r, p8_ref[...], preferred_element_type=F32)
        hpre = (num_ref[0] + num_ref[1]) * rex + b1_ref[...]
        hh = jnp.where(hpre > 0, hpre, jnp.exp(hpre) - 1.0)
        xw_ref[...] = jnp.dot(hh, w_ref[...], preferred_element_type=F32)
        ta_ref[...] = jnp.dot(hh, wa_ref[...], preferred_element_type=F32)

    return pl.pallas_call(
        body,
        out_shape=(jax.ShapeDtypeStruct((n, 128), F32),
                   jax.ShapeDtypeStruct((n, 16), F32)),
    )(num, den, b1, w2, w_a, p8)


def _final_tc(num, den, b2):
    n = num.shape[1]

    def body(num_ref, den_ref, b2_ref, o_ref):
        d = den_ref[0, :, :1] + den_ref[1, :, :1]
        o = (num_ref[0] + num_ref[1]) / (d + 1e-16) + b2_ref[...]
        m = jnp.max(o, axis=1, keepdims=True)
        z = o - m
        lse = jnp.log(jnp.sum(jnp.exp(z), axis=1, keepdims=True))
        o_ref[...] = z - lse

    return pl.pallas_call(
        body,
        out_shape=jax.ShapeDtypeStruct((n, 128), F32),
    )(num, den, b2)


def kernel(x, edge_index, W1, att_src1, att_dst1, b1, W2, att_src2,
           att_dst2, b2):
    n = x.shape[0]
    heads, hid = att_src1.shape
    eix = edge_index.reshape(2, -1, _C)

    eye = jnp.eye(heads, dtype=F32)
    a_s = (eye[:, None, :] * att_src1[:, :, None]).reshape(heads * hid, heads)
    a_d = (eye[:, None, :] * att_dst1[:, :, None]).reshape(heads * hid, heads)
    w1_a = jnp.concatenate([W1 @ a_s, W1 @ a_d], axis=1)
    w2_a = jnp.concatenate(
        [jnp.tile((W2 @ att_src2[0])[:, None], (1, 8)),
         jnp.tile((W2 @ att_dst2[0])[:, None], (1, 8))], axis=1)
    p8 = (jnp.arange(128)[None, :] // hid
          == jnp.arange(heads)[:, None]).astype(F32)
    zeros_hbm = jnp.zeros((n, 144), F32)

    xw1, ta1 = _prep_tc(x, W1, w1_a)
    num1, den1 = _edge_pass(xw1, ta1, eix, zeros_hbm)
    xw2, ta2 = _mid_tc(num1, den1, b1.reshape(1, -1), W2, w2_a, p8)
    num2, den2 = _edge_pass(xw2, ta2, eix, zeros_hbm)
    return _final_tc(num2, den2, b2.reshape(1, -1))
